# baseline (speedup 1.0000x reference)
import functools

import jax
import jax.numpy as jnp
from jax import lax
from jax.experimental import pallas as pl
from jax.experimental.pallas import tpu as pltpu
from jax.experimental.pallas import tpu_sc as plsc

N, C, H, W = 16, 3, 512, 512
PS = H * W
NW = 32
E = PS // NW
NV = E // 16

N_TC = 11
N_SC = N - N_TC

EPS = 1e-6
K1 = 0.6931471805599453 / (1 << 23)
K2 = 127.0 * 0.6931471805599453
P0 = 0.000925032111305707 - K2
P1 = 0.2866062324149039
P2 = -0.3935358023019213
P3 = 0.10668473260369027

MANT_MASK = 0x007FFFFF
ONE_BITS = 0x3F800000


R = 16


@functools.partial(
    pl.kernel,
    mesh=plsc.VectorSubcoreMesh(core_axis_name="c", subcore_axis_name="s"),
    out_type=[
        jax.ShapeDtypeStruct((NW, 16), jnp.float32),
        jax.ShapeDtypeStruct((NW, 16), jnp.float32),
    ],
    compiler_params=pltpu.CompilerParams(
        use_tc_tiling_on_sc=True, skip_device_barrier=True),
    scratch_types=[
        pltpu.VMEM((R, W), jnp.float32),
        pltpu.VMEM((R, W), jnp.float32),
        pltpu.VMEM((R, W), jnp.float32),
        pltpu.VMEM((R, W), jnp.float32),
        pltpu.VMEM((R, W), jnp.float32),
        pltpu.VMEM((R, W), jnp.float32),
        pltpu.VMEM((R, W), jnp.int32),
        pltpu.VMEM((R, W), jnp.int32),
        pltpu.VMEM((16,), jnp.float32),
        pltpu.VMEM((16,), jnp.float32),
        pltpu.SemaphoreType.DMA,
        pltpu.SemaphoreType.DMA,
    ],
)
def _nll_sc(t_hbm, m_hbm, s_hbm, k_hbm, out_l, out_c,
            tA, mA, sA, tB, mB, sB, kA, kB, al_v, ac_v,
            semA, semB):
    cid = lax.axis_index("c")
    sid = lax.axis_index("s")
    wid = sid * 2 + cid
    row0 = wid * R

    bufs = ((tA, mA, sA, kA, semA), (tB, mB, sB, kB, semB))
    S = N_SC * C

    def fire(s, p):
        tb, mb, sb, kb, sem = bufs[p]
        n = N_TC + s // C
        c = s % C
        pltpu.async_copy(t_hbm.at[n, c, pl.ds(row0, R), :], tb, sem)
        pltpu.async_copy(m_hbm.at[n, c, pl.ds(row0, R), :], mb, sem)
        pltpu.async_copy(s_hbm.at[n, c, pl.ds(row0, R), :], sb, sem)
        pltpu.async_copy(k_hbm.at[n, 0, pl.ds(row0, R), :], kb, sem)

    def drain(p):
        tb, mb, sb, kb, sem = bufs[p]
        dummy = t_hbm.at[0, 0, pl.ds(0, R), :]
        pltpu.make_async_copy(dummy, tb, sem).wait()
        pltpu.make_async_copy(dummy, mb, sem).wait()
        pltpu.make_async_copy(dummy, sb, sem).wait()
        pltpu.make_async_copy(k_hbm.at[0, 0, pl.ds(0, R), :], kb, sem).wait()

    def compute(p, carry):
        tb, mb, sb, kb, _ = bufs[p]

        def inner(i, ic):
            al, ac = ic
            r = i >> 5
            sl = pl.ds((i & 31) * 16, 16)
            t = tb[r, sl]
            mu = mb[r, sl]
            s = sb[r, sl]
            mf = kb[r, sl].astype(jnp.float32)
            d = mu - t
            var = s * s
            cv = jnp.maximum(var, EPS)
            q = (d * d) / cv
            bits = lax.bitcast_convert_type(cv, jnp.int32)
            lo = bits.astype(jnp.float32) * K1
            mb_ = jnp.bitwise_or(jnp.bitwise_and(bits, MANT_MASK), ONE_BITS)
            z = lax.bitcast_convert_type(mb_, jnp.float32) - 1.0
            poly = ((P3 * z + P2) * z + P1) * z + P0
            al = al + (lo + poly + q) * mf
            ac = ac + mf
            return (al, ac)

        return lax.fori_loop(0, NV, inner, carry)

    fire(0, 0)
    fire(1, 1)

    def pair(j, carry):
        drain(0)
        carry = compute(0, carry)

        @pl.when(2 * j + 2 < S)
        def _():
            fire(2 * j + 2, 0)

        drain(1)
        carry = compute(1, carry)

        @pl.when(2 * j + 3 < S)
        def _():
            fire(2 * j + 3, 1)

        return carry

    zero = jnp.zeros((16,), jnp.float32)
    carry = lax.fori_loop(0, S // 2, pair, (zero, zero))
    if S % 2 == 1:
        drain(0)
        carry = compute(0, carry)
    accl, accc = carry

    al_v[...] = accl
    ac_v[...] = accc
    pltpu.sync_copy(al_v, out_l.at[wid])
    pltpu.sync_copy(ac_v, out_c.at[wid])


def _tc_body(t_ref, m_ref, s_ref, k_ref, out_l, out_c):
    p = pl.program_id(0)

    @pl.when(p == 0)
    def _():
        out_l[...] = jnp.zeros_like(out_l)
        out_c[...] = jnp.zeros_like(out_c)

    mf = k_ref[0, 0].astype(jnp.float32)
    acc = jnp.zeros((H, W), jnp.float32)
    for c in range(C):
        t = t_ref[0, c]
        mu = m_ref[0, c]
        s = s_ref[0, c]
        var = s * s
        cv = jnp.maximum(var, EPS)
        d = mu - t
        acc = acc + (jnp.log(cv) + (d * d) / cv)
    loss = acc * mf
    out_l[...] += jnp.sum(loss.reshape(H // 8, 8, W), axis=0)
    out_c[...] += jnp.sum(mf.reshape(H // 8, 8, W), axis=0)


_nll_tc = pl.pallas_call(
    _tc_body,
    grid=(N_TC,),
    in_specs=[
        pl.BlockSpec((1, C, H, W), lambda p: (p, 0, 0, 0)),
        pl.BlockSpec((1, C, H, W), lambda p: (p, 0, 0, 0)),
        pl.BlockSpec((1, C, H, W), lambda p: (p, 0, 0, 0)),
        pl.BlockSpec((1, 1, H, W), lambda p: (p, 0, 0, 0)),
    ],
    out_specs=[
        pl.BlockSpec((8, W), lambda p: (0, 0)),
        pl.BlockSpec((8, W), lambda p: (0, 0)),
    ],
    out_shape=[
        jax.ShapeDtypeStruct((8, W), jnp.float32),
        jax.ShapeDtypeStruct((8, W), jnp.float32),
    ],
)


@jax.jit
def kernel(target, mean, std, mask):
    sc_l, sc_c = _nll_sc(target, mean, std, mask)
    tc_l, tc_c = _nll_tc(target, mean, std, mask)
    num = 0.5 * (jnp.sum(sc_l) + jnp.sum(tc_l))
    den = jnp.sum(sc_c) * (1.0 / C) + jnp.sum(tc_c)
    return num / den


# speedup vs baseline: 1.0447x; 1.0447x over previous
import functools

import jax
import jax.numpy as jnp
from jax import lax
from jax.experimental import pallas as pl
from jax.experimental.pallas import tpu as pltpu
from jax.experimental.pallas import tpu_sc as plsc

N, C, H, W = 16, 3, 512, 512
PS = H * W
NW = 32
E = PS // NW
NV = E // 16

N_TC = 10
N_SC = N - N_TC

EPS = 1e-6
K1 = 0.6931471805599453 / (1 << 23)
K2 = 127.0 * 0.6931471805599453
P0 = 0.000925032111305707 - K2
P1 = 0.2866062324149039
P2 = -0.3935358023019213
P3 = 0.10668473260369027

MANT_MASK = 0x007FFFFF
ONE_BITS = 0x3F800000


R = 16


@functools.partial(
    pl.kernel,
    mesh=plsc.VectorSubcoreMesh(core_axis_name="c", subcore_axis_name="s"),
    out_type=[
        jax.ShapeDtypeStruct((NW, 16), jnp.float32),
        jax.ShapeDtypeStruct((NW, 16), jnp.float32),
    ],
    compiler_params=pltpu.CompilerParams(use_tc_tiling_on_sc=True),
    scratch_types=[
        pltpu.VMEM((R, W), jnp.float32),
        pltpu.VMEM((R, W), jnp.float32),
        pltpu.VMEM((R, W), jnp.float32),
        pltpu.VMEM((R, W), jnp.float32),
        pltpu.VMEM((R, W), jnp.float32),
        pltpu.VMEM((R, W), jnp.float32),
        pltpu.VMEM((R, W), jnp.int32),
        pltpu.VMEM((R, W), jnp.int32),
        pltpu.VMEM((16,), jnp.float32),
        pltpu.VMEM((16,), jnp.float32),
        pltpu.SemaphoreType.DMA,
        pltpu.SemaphoreType.DMA,
        pltpu.SemaphoreType.DMA,
        pltpu.SemaphoreType.DMA,
    ],
)
def _nll_sc(t_hbm, m_hbm, s_hbm, k_hbm, out_l, out_c,
            tA, mA, sA, tB, mB, sB, kA, kB, al_v, ac_v,
            semA, semB, msemA, msemB):
    cid = lax.axis_index("c")
    sid = lax.axis_index("s")
    wid = sid * 2 + cid
    row0 = wid * R

    bufs = ((tA, mA, sA, semA), (tB, mB, sB, semB))
    kbufs = ((kA, msemA), (kB, msemB))

    def fire(n, c, p):
        tb, mb, sb, sem = bufs[p]
        pltpu.async_copy(t_hbm.at[n, c, pl.ds(row0, R), :], tb, sem)
        pltpu.async_copy(m_hbm.at[n, c, pl.ds(row0, R), :], mb, sem)
        pltpu.async_copy(s_hbm.at[n, c, pl.ds(row0, R), :], sb, sem)

    def drain(p):
        tb, mb, sb, sem = bufs[p]
        dummy = t_hbm.at[0, 0, pl.ds(0, R), :]
        pltpu.make_async_copy(dummy, tb, sem).wait()
        pltpu.make_async_copy(dummy, mb, sem).wait()
        pltpu.make_async_copy(dummy, sb, sem).wait()

    def fire_mask(n, p):
        kb, msem = kbufs[p]
        pltpu.async_copy(k_hbm.at[n, 0, pl.ds(row0, R), :], kb, msem)

    def drain_mask(p):
        kb, msem = kbufs[p]
        pltpu.make_async_copy(k_hbm.at[0, 0, pl.ds(0, R), :], kb, msem).wait()

    def compute(p, kp, first, carry):
        tb, mb, sb, _ = bufs[p]
        kb, _ = kbufs[kp]

        def inner(i, ic):
            al, ac = ic
            r = i >> 5
            sl = pl.ds((i & 31) * 16, 16)
            t = tb[r, sl]
            mu = mb[r, sl]
            s = sb[r, sl]
            mf = kb[r, sl].astype(jnp.float32)
            d = mu - t
            var = s * s
            cv = jnp.maximum(var, EPS)
            q = (d * d) / cv
            bits = lax.bitcast_convert_type(cv, jnp.int32)
            lo = bits.astype(jnp.float32) * K1
            mb_ = jnp.bitwise_or(jnp.bitwise_and(bits, MANT_MASK), ONE_BITS)
            z = lax.bitcast_convert_type(mb_, jnp.float32) - 1.0
            poly = ((P3 * z + P2) * z + P1) * z + P0
            al = al + (lo + poly + q) * mf
            if first:
                ac = ac + mf
            return (al, ac)

        return lax.fori_loop(0, NV, inner, carry)

    def one_image(n, e, carry, last):
        o = 1 - e
        fire(n, 1, o)
        drain_mask(e)
        drain(e)
        carry = compute(e, e, True, carry)
        if not last:
            fire_mask(n + 1, o)
        fire(n, 2, e)
        drain(o)
        carry = compute(o, e, False, carry)
        if not last:
            fire(n + 1, 0, o)
        drain(e)
        carry = compute(e, e, False, carry)
        return carry

    fire_mask(N_TC, 0)
    fire(N_TC, 0, 0)

    zero = jnp.zeros((16,), jnp.float32)
    carry = (zero, zero)
    for i in range(N_SC):
        carry = one_image(N_TC + i, i % 2, carry, last=(i == N_SC - 1))
    accl, accc = carry

    al_v[...] = accl
    ac_v[...] = accc
    pltpu.sync_copy(al_v, out_l.at[wid])
    pltpu.sync_copy(ac_v, out_c.at[wid])


def _tc_body(t_ref, m_ref, s_ref, k_ref, out_l, out_c):
    p = pl.program_id(0)

    @pl.when(p == 0)
    def _():
        out_l[...] = jnp.zeros_like(out_l)
        out_c[...] = jnp.zeros_like(out_c)

    mf = k_ref[0, 0].astype(jnp.float32)
    acc = jnp.zeros((H, W), jnp.float32)
    for c in range(C):
        t = t_ref[0, c]
        mu = m_ref[0, c]
        s = s_ref[0, c]
        var = s * s
        cv = jnp.maximum(var, EPS)
        d = mu - t
        acc = acc + (jnp.log(cv) + (d * d) / cv)
    loss = acc * mf
    out_l[...] += jnp.sum(loss.reshape(H // 8, 8, W), axis=0)
    out_c[...] += jnp.sum(mf.reshape(H // 8, 8, W), axis=0)


_nll_tc = pl.pallas_call(
    _tc_body,
    grid=(N_TC,),
    in_specs=[
        pl.BlockSpec((1, C, H, W), lambda p: (p, 0, 0, 0)),
        pl.BlockSpec((1, C, H, W), lambda p: (p, 0, 0, 0)),
        pl.BlockSpec((1, C, H, W), lambda p: (p, 0, 0, 0)),
        pl.BlockSpec((1, 1, H, W), lambda p: (p, 0, 0, 0)),
    ],
    out_specs=[
        pl.BlockSpec((8, W), lambda p: (0, 0)),
        pl.BlockSpec((8, W), lambda p: (0, 0)),
    ],
    out_shape=[
        jax.ShapeDtypeStruct((8, W), jnp.float32),
        jax.ShapeDtypeStruct((8, W), jnp.float32),
    ],
)


@jax.jit
def kernel(target, mean, std, mask):
    sc_l, sc_c = _nll_sc(target, mean, std, mask)
    tc_l, tc_c = _nll_tc(target, mean, std, mask)
    num = 0.5 * (jnp.sum(sc_l) + jnp.sum(tc_l))
    den = jnp.sum(sc_c) + jnp.sum(tc_c)
    return num / den


# speedup vs baseline: 1.0914x; 1.0448x over previous
import functools

import jax
import jax.numpy as jnp
from jax import lax
from jax.experimental import pallas as pl
from jax.experimental.pallas import tpu as pltpu
from jax.experimental.pallas import tpu_sc as plsc

N, C, H, W = 16, 3, 512, 512
PS = H * W
NW = 32
E = PS // NW
NV = E // 16

N_TC = 10
N_SC = N - N_TC

EPS = 1e-6
K1 = 0.6931471805599453 / (1 << 23)
K2 = 127.0 * 0.6931471805599453
P0 = 0.000925032111305707 - K2
P1 = 0.2866062324149039
P2 = -0.3935358023019213
P3 = 0.10668473260369027

MANT_MASK = 0x007FFFFF
ONE_BITS = 0x3F800000


R = 16


@functools.partial(
    pl.kernel,
    mesh=plsc.VectorSubcoreMesh(core_axis_name="c", subcore_axis_name="s"),
    out_type=[
        jax.ShapeDtypeStruct((NW, 16), jnp.float32),
        jax.ShapeDtypeStruct((NW, 16), jnp.float32),
    ],
    compiler_params=pltpu.CompilerParams(use_tc_tiling_on_sc=True),
    scratch_types=[
        pltpu.VMEM((R, W), jnp.float32),
        pltpu.VMEM((R, W), jnp.float32),
        pltpu.VMEM((R, W), jnp.float32),
        pltpu.VMEM((R, W), jnp.float32),
        pltpu.VMEM((R, W), jnp.float32),
        pltpu.VMEM((R, W), jnp.float32),
        pltpu.VMEM((R, W), jnp.int32),
        pltpu.VMEM((R, W), jnp.int32),
        pltpu.VMEM((16,), jnp.float32),
        pltpu.VMEM((16,), jnp.float32),
        pltpu.SemaphoreType.DMA,
        pltpu.SemaphoreType.DMA,
        pltpu.SemaphoreType.DMA,
        pltpu.SemaphoreType.DMA,
    ],
)
def _nll_sc(t_hbm, m_hbm, s_hbm, k_hbm, out_l, out_c,
            tA, mA, sA, tB, mB, sB, kA, kB, al_v, ac_v,
            semA, semB, msemA, msemB):
    cid = lax.axis_index("c")
    sid = lax.axis_index("s")
    wid = sid * 2 + cid
    row0 = wid * R

    bufs = ((tA, mA, sA, semA), (tB, mB, sB, semB))
    kbufs = ((kA, msemA), (kB, msemB))

    def fire(n, c, p):
        tb, mb, sb, sem = bufs[p]
        pltpu.async_copy(t_hbm.at[n, c, pl.ds(row0, R), :], tb, sem)
        pltpu.async_copy(m_hbm.at[n, c, pl.ds(row0, R), :], mb, sem)
        pltpu.async_copy(s_hbm.at[n, c, pl.ds(row0, R), :], sb, sem)

    def drain(p):
        tb, mb, sb, sem = bufs[p]
        dummy = t_hbm.at[0, 0, pl.ds(0, R), :]
        pltpu.make_async_copy(dummy, tb, sem).wait()
        pltpu.make_async_copy(dummy, mb, sem).wait()
        pltpu.make_async_copy(dummy, sb, sem).wait()

    def fire_mask(n, p):
        kb, msem = kbufs[p]
        pltpu.async_copy(k_hbm.at[n, 0, pl.ds(row0, R), :], kb, msem)

    def drain_mask(p):
        kb, msem = kbufs[p]
        pltpu.make_async_copy(k_hbm.at[0, 0, pl.ds(0, R), :], kb, msem).wait()

    def compute(p, kp, first, carry):
        tb, mb, sb, _ = bufs[p]
        kb, _ = kbufs[kp]

        def inner(i, ic):
            al, ac = ic
            r = i >> 5
            sl = pl.ds((i & 31) * 16, 16)
            t = tb[r, sl]
            mu = mb[r, sl]
            s = sb[r, sl]
            mf = kb[r, sl].astype(jnp.float32)
            d = mu - t
            var = s * s
            cv = jnp.maximum(var, EPS)
            q = (d * d) / cv
            bits = lax.bitcast_convert_type(cv, jnp.int32)
            lo = bits.astype(jnp.float32) * K1
            mb_ = jnp.bitwise_or(jnp.bitwise_and(bits, MANT_MASK), ONE_BITS)
            z = lax.bitcast_convert_type(mb_, jnp.float32) - 1.0
            poly = ((P3 * z + P2) * z + P1) * z + P0
            al = al + (lo + poly + q) * mf
            if first:
                ac = ac + mf
            return (al, ac)

        return lax.fori_loop(0, NV, inner, carry)

    def one_image(n, e, carry, last):
        o = 1 - e
        fire(n, 1, o)
        drain_mask(e)
        drain(e)
        carry = compute(e, e, True, carry)
        if not last:
            fire_mask(n + 1, o)
        fire(n, 2, e)
        drain(o)
        carry = compute(o, e, False, carry)
        if not last:
            fire(n + 1, 0, o)
        drain(e)
        carry = compute(e, e, False, carry)
        return carry

    fire_mask(N_TC, 0)
    fire(N_TC, 0, 0)

    zero = jnp.zeros((16,), jnp.float32)
    carry = (zero, zero)
    for i in range(N_SC):
        carry = one_image(N_TC + i, i % 2, carry, last=(i == N_SC - 1))
    accl, accc = carry

    al_v[...] = accl
    ac_v[...] = accc
    pltpu.sync_copy(al_v, out_l.at[wid])
    pltpu.sync_copy(ac_v, out_c.at[wid])


def _tc_body(t_ref, m_ref, s_ref, k_ref, out_l, out_c, acc_l, acc_c):
    p = pl.program_id(0)

    @pl.when(p == 0)
    def _():
        acc_l[...] = jnp.zeros_like(acc_l)
        acc_c[...] = jnp.zeros_like(acc_c)

    mf = k_ref[0, 0].astype(jnp.float32)
    acc = jnp.zeros((H, W), jnp.float32)
    for c in range(C):
        t = t_ref[0, c]
        mu = m_ref[0, c]
        s = s_ref[0, c]
        var = s * s
        cv = jnp.maximum(var, EPS)
        d = mu - t
        acc = acc + (jnp.log(cv) + (d * d) / cv)
    loss = acc * mf
    acc_l[...] += jnp.sum(loss.reshape(H // 8, 8, W), axis=0)
    acc_c[...] += jnp.sum(mf.reshape(H // 8, 8, W), axis=0)

    @pl.when(p == N_TC - 1)
    def _():
        out_l[0, 0] = jnp.sum(acc_l[...])
        out_c[0, 0] = jnp.sum(acc_c[...])


_nll_tc = pl.pallas_call(
    _tc_body,
    grid=(N_TC,),
    in_specs=[
        pl.BlockSpec((1, C, H, W), lambda p: (p, 0, 0, 0)),
        pl.BlockSpec((1, C, H, W), lambda p: (p, 0, 0, 0)),
        pl.BlockSpec((1, C, H, W), lambda p: (p, 0, 0, 0)),
        pl.BlockSpec((1, 1, H, W), lambda p: (p, 0, 0, 0)),
    ],
    out_specs=[
        pl.BlockSpec(memory_space=pltpu.SMEM),
        pl.BlockSpec(memory_space=pltpu.SMEM),
    ],
    out_shape=[
        jax.ShapeDtypeStruct((1, 1), jnp.float32),
        jax.ShapeDtypeStruct((1, 1), jnp.float32),
    ],
    scratch_shapes=[
        pltpu.VMEM((8, W), jnp.float32),
        pltpu.VMEM((8, W), jnp.float32),
    ],
)


@jax.jit
def kernel(target, mean, std, mask):
    sc_l, sc_c = _nll_sc(target, mean, std, mask)
    tc_l, tc_c = _nll_tc(target, mean, std, mask)
    sc_sums = jnp.sum(jnp.stack([sc_l, sc_c]), axis=(1, 2))
    num = 0.5 * (sc_sums[0] + tc_l[0, 0])
    den = sc_sums[1] + tc_c[0, 0]
    return num / den
